# baseline (speedup 1.0000x reference)
import jax
import jax.numpy as jnp
from jax import lax
from jax.experimental import pallas as pl
from jax.experimental.pallas import tpu as pltpu
from jax.experimental.pallas import tpu_sc as plsc

_B = 4
_D = 256
_H = 32
_W = 32
_L = 16


def _pos_body(row_hbm, col_hbm, out_hbm, idx, bufc, bufr, sem_in, sem_out):
    cid = lax.axis_index("c")
    sid = lax.axis_index("s")
    y = cid * (_H // 2) + sid

    cp_col = pltpu.async_copy(col_hbm.at[pl.ds(0, _W)], bufc, sem_in)
    yv = jnp.full((_L,), y, jnp.int32)
    idx[pl.ds(0, _L)] = yv
    idx[pl.ds(_L, _L)] = yv
    cp_row = pltpu.async_copy(row_hbm.at[idx], bufr, sem_in)

    cp_col.wait()
    outs = [
        pltpu.async_copy(bufc, out_hbm.at[b, y, :, pl.ds(0, _D)], sem_out)
        for b in range(_B)
    ]
    cp_row.wait()
    outs += [
        pltpu.async_copy(bufr, out_hbm.at[b, y, :, pl.ds(_D, _D)], sem_out)
        for b in range(_B)
    ]
    for cp in outs:
        cp.wait()


def kernel(img, mask, row_embed, col_embed):
    del img, mask
    mesh = plsc.VectorSubcoreMesh(core_axis_name="c", subcore_axis_name="s")
    fn = pl.kernel(
        _pos_body,
        mesh=mesh,
        out_type=jax.ShapeDtypeStruct((_B, _H, _W, 2 * _D), jnp.float32),
        scratch_types=[
            pltpu.VMEM((_W,), jnp.int32),
            pltpu.VMEM((_W, _D), jnp.float32),
            pltpu.VMEM((_W, _D), jnp.float32),
            pltpu.SemaphoreType.DMA,
            pltpu.SemaphoreType.DMA,
        ],
        compiler_params=pltpu.CompilerParams(use_tc_tiling_on_sc=True),
    )
    out_t = fn(row_embed, col_embed)
    return jnp.transpose(out_t, (0, 3, 1, 2))


# speedup vs baseline: 1.0906x; 1.0906x over previous
import jax
import jax.numpy as jnp
from jax import lax
from jax.experimental import pallas as pl
from jax.experimental.pallas import tpu as pltpu
from jax.experimental.pallas import tpu_sc as plsc

_B = 4
_D = 256
_H = 32
_W = 32
_L = 16


def _pos_body(row_hbm, col_hbm, out_hbm, idx, buf, sem):
    cid = lax.axis_index("c")
    sid = lax.axis_index("s")
    y = cid * (_H // 2) + sid
    yv = jnp.full((_L,), y, jnp.int32)
    idx[pl.ds(0, _L)] = yv
    idx[pl.ds(_L, _L)] = yv

    cp_col = pltpu.async_copy(
        col_hbm.at[pl.ds(0, _W)], buf.at[:, pl.ds(0, _D)], sem
    )
    cp_row = pltpu.async_copy(
        row_hbm.at[idx], buf.at[:, pl.ds(_D, _D)], sem
    )
    cp_col.wait()
    cp_row.wait()

    copies = [
        pltpu.async_copy(buf, out_hbm.at[b, y], sem) for b in range(_B)
    ]
    for cp in copies:
        cp.wait()


def kernel(img, mask, row_embed, col_embed):
    del img, mask
    mesh = plsc.VectorSubcoreMesh(core_axis_name="c", subcore_axis_name="s")
    fn = pl.kernel(
        _pos_body,
        mesh=mesh,
        out_type=jax.ShapeDtypeStruct((_B, _H, _W, 2 * _D), jnp.float32),
        scratch_types=[
            pltpu.VMEM((_W,), jnp.int32),
            pltpu.VMEM((_W, 2 * _D), jnp.float32),
            pltpu.SemaphoreType.DMA,
        ],
        compiler_params=pltpu.CompilerParams(
            use_tc_tiling_on_sc=True, skip_device_barrier=True
        ),
    )
    out_t = fn(row_embed, col_embed)
    return jnp.transpose(out_t, (0, 3, 1, 2))
